# baseline (speedup 1.0000x reference)
import jax
import jax.numpy as jnp
from jax.experimental import pallas as pl
from jax.experimental.pallas import tpu as pltpu

_TILE = 1024
_LANES = 128


def _fused_body(state_ref, action_ref, values_ref, out_ref):
    s = state_ref[...].astype(jnp.bfloat16)
    v = values_ref[...].astype(jnp.bfloat16)
    q = jnp.dot(s, v, preferred_element_type=jnp.float32)
    r = (action_ref[...] * q).reshape(_TILE // _LANES, _LANES, -1)
    out_ref[...] = jnp.sum(r, axis=2)


def kernel(state, action, values):
    batch, state_size = state.shape
    action_size = action.shape[1]
    kpad = (state_size + _LANES - 1) // _LANES * _LANES
    state_pad = jnp.pad(state, ((0, 0), (0, kpad - state_size)))
    values_pad = jnp.pad(values, ((0, kpad - state_size), (0, 0)))
    sub = _TILE // _LANES
    grid = (batch // _TILE,)

    out = pl.pallas_call(
        _fused_body,
        grid=grid,
        in_specs=[
            pl.BlockSpec((_TILE, kpad), lambda i: (i, 0)),
            pl.BlockSpec((_TILE, action_size), lambda i: (i, 0)),
            pl.BlockSpec((kpad, action_size), lambda i: (0, 0)),
        ],
        out_specs=pl.BlockSpec((sub, _LANES), lambda i: (i, 0)),
        out_shape=jax.ShapeDtypeStruct((batch // _LANES, _LANES), jnp.float32),
        compiler_params=pltpu.CompilerParams(
            dimension_semantics=("arbitrary",),
            allow_input_fusion=[True, False, False],
        ),
    )(state_pad, action, values_pad)
    return out.reshape(batch, 1)


# speedup vs baseline: 1.0748x; 1.0748x over previous
import jax
import jax.numpy as jnp
from jax.experimental import pallas as pl
from jax.experimental.pallas import tpu as pltpu

_TILE = 1024
_LANES = 128


def _fused_body(state_ref, action_ref, values_ref, out_ref):
    s = state_ref[...].astype(jnp.bfloat16)
    v = values_ref[...].astype(jnp.bfloat16)
    q = jnp.dot(s, v, preferred_element_type=jnp.float32)
    r = (action_ref[...] * q).reshape(_TILE // _LANES, _LANES, -1)
    out_ref[...] = jnp.sum(r, axis=2)


def kernel(state, action, values):
    batch, state_size = state.shape
    action_size = action.shape[1]
    sub = _TILE // _LANES
    grid = (batch // _TILE,)
    out = pl.pallas_call(
        _fused_body,
        grid=grid,
        in_specs=[
            pl.BlockSpec((_TILE, state_size), lambda i: (i, 0)),
            pl.BlockSpec((_TILE, action_size), lambda i: (i, 0)),
            pl.BlockSpec((state_size, action_size), lambda i: (0, 0)),
        ],
        out_specs=pl.BlockSpec((sub, _LANES), lambda i: (i, 0)),
        out_shape=jax.ShapeDtypeStruct((batch // _LANES, _LANES), jnp.float32),
        compiler_params=pltpu.CompilerParams(
            dimension_semantics=("parallel",),
        ),
    )(state, action, values)
    return out.reshape(batch, 1)


# speedup vs baseline: 1.1092x; 1.0321x over previous
import jax
import jax.numpy as jnp
from jax.experimental import pallas as pl
from jax.experimental.pallas import tpu as pltpu

_TILE = 2048
_LANES = 128


def _fused_body(state_ref, action_ref, values_ref, out_ref):
    s = state_ref[...].astype(jnp.bfloat16)
    v = values_ref[...].astype(jnp.bfloat16)
    q = jnp.dot(s, v, preferred_element_type=jnp.float32)
    r = (action_ref[...] * q).reshape(_TILE // _LANES, _LANES, -1)
    out_ref[...] = jnp.sum(r, axis=2)


def kernel(state, action, values):
    batch, state_size = state.shape
    action_size = action.shape[1]
    sub = _TILE // _LANES
    grid = (batch // _TILE,)
    out = pl.pallas_call(
        _fused_body,
        grid=grid,
        in_specs=[
            pl.BlockSpec((_TILE, state_size), lambda i: (i, 0)),
            pl.BlockSpec((_TILE, action_size), lambda i: (i, 0)),
            pl.BlockSpec((state_size, action_size), lambda i: (0, 0)),
        ],
        out_specs=pl.BlockSpec((sub, _LANES), lambda i: (i, 0)),
        out_shape=jax.ShapeDtypeStruct((batch // _LANES, _LANES), jnp.float32),
        compiler_params=pltpu.CompilerParams(
            dimension_semantics=("parallel",),
        ),
    )(state, action, values)
    return out.reshape(batch, 1)
